# baseline (speedup 1.0000x reference)
import jax
import jax.numpy as jnp
from jax.experimental import pallas as pl
from jax.experimental.pallas import tpu as pltpu

D_MODEL = 1024
D_FF = 2048
NUM_EXPERTS = 8
N_TOKENS = 2048
BT = 256


def _moe_kernel(x_ref, W1_ref, b1_ref, W2_ref, b2_ref, Wg_ref, bg_ref,
                out_ref):
    e = pl.program_id(0)
    t = pl.program_id(1)
    x = x_ref[...]

    logits = jnp.dot(x.astype(jnp.bfloat16), Wg_ref[...].astype(jnp.bfloat16),
                     preferred_element_type=jnp.float32) + bg_ref[0]
    ii = jax.lax.broadcasted_iota(jnp.int32, logits.shape, 1)
    m1 = jnp.max(logits, axis=-1, keepdims=True)
    i1 = jnp.min(jnp.where(logits == m1, ii, NUM_EXPERTS),
                 axis=-1, keepdims=True)
    logits2 = jnp.where(ii == i1, -jnp.inf, logits)
    m2 = jnp.max(logits2, axis=-1, keepdims=True)
    i2 = jnp.min(jnp.where(logits2 == m2, ii, NUM_EXPERTS),
                 axis=-1, keepdims=True)
    g1 = jax.nn.sigmoid(m1 - m2)
    g2 = 1.0 - g1
    w_e = jnp.where(i1 == e, g1, 0.0) + jnp.where(i2 == e, g2, 0.0)

    xb = x.astype(jnp.bfloat16)
    h = (jnp.dot(xb, W1_ref[0], preferred_element_type=jnp.float32)
         + b1_ref[0, 0])
    hb = jnp.maximum(h, 0.0).astype(jnp.bfloat16)
    o = (jnp.dot(hb, W2_ref[0], preferred_element_type=jnp.float32)
         + b2_ref[0, 0])
    contrib = o * w_e

    sl = pl.ds(t * BT, BT)

    @pl.when(e == 0)
    def _():
        out_ref[sl, :] = contrib

    @pl.when(e != 0)
    def _():
        out_ref[sl, :] = out_ref[sl, :] + contrib


def _cast_kernel(W1_ref, W2_ref, W1b_ref, W2b_ref):
    W1b_ref[...] = W1_ref[...].astype(jnp.bfloat16)
    W2b_ref[...] = W2_ref[...].astype(jnp.bfloat16)


def _precast_weights(W1, W2):
    return pl.pallas_call(
        _cast_kernel,
        grid=(NUM_EXPERTS,),
        in_specs=[
            pl.BlockSpec((1, D_MODEL, D_FF), lambda e: (e, 0, 0)),
            pl.BlockSpec((1, D_FF, D_MODEL), lambda e: (e, 0, 0)),
        ],
        out_specs=[
            pl.BlockSpec((1, D_MODEL, D_FF), lambda e: (e, 0, 0)),
            pl.BlockSpec((1, D_FF, D_MODEL), lambda e: (e, 0, 0)),
        ],
        out_shape=[
            jax.ShapeDtypeStruct((NUM_EXPERTS, D_MODEL, D_FF), jnp.bfloat16),
            jax.ShapeDtypeStruct((NUM_EXPERTS, D_FF, D_MODEL), jnp.bfloat16),
        ],
    )(W1, W2)


def kernel(x, W1, b1, W2, b2, Wg, bg):
    W1b, W2b = _precast_weights(W1, W2)
    bg2 = bg.reshape(1, NUM_EXPERTS)
    b1r = b1.reshape(NUM_EXPERTS, 1, D_FF)
    b2r = b2.reshape(NUM_EXPERTS, 1, D_MODEL)
    grid = (NUM_EXPERTS, N_TOKENS // BT)
    return pl.pallas_call(
        _moe_kernel,
        grid=grid,
        in_specs=[
            pl.BlockSpec((BT, D_MODEL), lambda e, t: (t, 0)),
            pl.BlockSpec((1, D_MODEL, D_FF), lambda e, t: (e, 0, 0)),
            pl.BlockSpec((1, 1, D_FF), lambda e, t: (e, 0, 0)),
            pl.BlockSpec((1, D_FF, D_MODEL), lambda e, t: (e, 0, 0)),
            pl.BlockSpec((1, 1, D_MODEL), lambda e, t: (e, 0, 0)),
            pl.BlockSpec((D_MODEL, NUM_EXPERTS), lambda e, t: (0, 0)),
            pl.BlockSpec((1, NUM_EXPERTS), lambda e, t: (0, 0)),
        ],
        out_specs=pl.BlockSpec((N_TOKENS, D_MODEL), lambda e, t: (0, 0)),
        out_shape=jax.ShapeDtypeStruct((N_TOKENS, D_MODEL), jnp.float32),
        compiler_params=pltpu.CompilerParams(
            dimension_semantics=("arbitrary", "arbitrary"),
        ),
    )(x, W1b, b1r, W2b, b2r, Wg, bg2)


# speedup vs baseline: 1.1936x; 1.1936x over previous
import jax
import jax.numpy as jnp
from jax.experimental import pallas as pl
from jax.experimental.pallas import tpu as pltpu

D_MODEL = 1024
D_FF = 2048
NUM_EXPERTS = 8
N_TOKENS = 2048
BT = 256


def _moe_kernel(x_ref, W1_ref, b1_ref, W2_ref, b2_ref, Wg_ref, bg_ref,
                out_ref):
    e = pl.program_id(0)
    t = pl.program_id(1)
    x = x_ref[...]

    logits = jnp.dot(x.astype(jnp.bfloat16), Wg_ref[...].astype(jnp.bfloat16),
                     preferred_element_type=jnp.float32) + bg_ref[0]
    ii = jax.lax.broadcasted_iota(jnp.int32, logits.shape, 1)
    m1 = jnp.max(logits, axis=-1, keepdims=True)
    i1 = jnp.min(jnp.where(logits == m1, ii, NUM_EXPERTS),
                 axis=-1, keepdims=True)
    logits2 = jnp.where(ii == i1, -jnp.inf, logits)
    m2 = jnp.max(logits2, axis=-1, keepdims=True)
    i2 = jnp.min(jnp.where(logits2 == m2, ii, NUM_EXPERTS),
                 axis=-1, keepdims=True)
    g1 = jax.nn.sigmoid(m1 - m2)
    g2 = 1.0 - g1
    w_e = jnp.where(i1 == e, g1, 0.0) + jnp.where(i2 == e, g2, 0.0)

    h = (jnp.dot(x, W1_ref[0], preferred_element_type=jnp.float32)
         + b1_ref[0, 0])
    hb = jnp.maximum(h, 0.0)
    o = (jnp.dot(hb, W2_ref[0], preferred_element_type=jnp.float32)
         + b2_ref[0, 0])
    contrib = o * w_e

    sl = pl.ds(t * BT, BT)

    @pl.when(e == 0)
    def _():
        out_ref[sl, :] = contrib

    @pl.when(e != 0)
    def _():
        out_ref[sl, :] = out_ref[sl, :] + contrib


def _cast_kernel(W1_ref, W2_ref, W1b_ref, W2b_ref):
    W1b_ref[...] = W1_ref[...].astype(jnp.bfloat16)
    W2b_ref[...] = W2_ref[...].astype(jnp.bfloat16)


def _precast_weights(W1, W2):
    return pl.pallas_call(
        _cast_kernel,
        grid=(NUM_EXPERTS,),
        in_specs=[
            pl.BlockSpec((1, D_MODEL, D_FF), lambda e: (e, 0, 0)),
            pl.BlockSpec((1, D_FF, D_MODEL), lambda e: (e, 0, 0)),
        ],
        out_specs=[
            pl.BlockSpec((1, D_MODEL, D_FF), lambda e: (e, 0, 0)),
            pl.BlockSpec((1, D_FF, D_MODEL), lambda e: (e, 0, 0)),
        ],
        out_shape=[
            jax.ShapeDtypeStruct((NUM_EXPERTS, D_MODEL, D_FF), jnp.bfloat16),
            jax.ShapeDtypeStruct((NUM_EXPERTS, D_FF, D_MODEL), jnp.bfloat16),
        ],
    )(W1, W2)


def kernel(x, W1, b1, W2, b2, Wg, bg):
    W1b, W2b = W1, W2
    bg2 = bg.reshape(1, NUM_EXPERTS)
    b1r = b1.reshape(NUM_EXPERTS, 1, D_FF)
    b2r = b2.reshape(NUM_EXPERTS, 1, D_MODEL)
    grid = (NUM_EXPERTS, N_TOKENS // BT)
    return pl.pallas_call(
        _moe_kernel,
        grid=grid,
        in_specs=[
            pl.BlockSpec((BT, D_MODEL), lambda e, t: (t, 0)),
            pl.BlockSpec((1, D_MODEL, D_FF), lambda e, t: (e, 0, 0)),
            pl.BlockSpec((1, 1, D_FF), lambda e, t: (e, 0, 0)),
            pl.BlockSpec((1, D_FF, D_MODEL), lambda e, t: (e, 0, 0)),
            pl.BlockSpec((1, 1, D_MODEL), lambda e, t: (e, 0, 0)),
            pl.BlockSpec((D_MODEL, NUM_EXPERTS), lambda e, t: (0, 0)),
            pl.BlockSpec((1, NUM_EXPERTS), lambda e, t: (0, 0)),
        ],
        out_specs=pl.BlockSpec((N_TOKENS, D_MODEL), lambda e, t: (0, 0)),
        out_shape=jax.ShapeDtypeStruct((N_TOKENS, D_MODEL), jnp.float32),
        compiler_params=pltpu.CompilerParams(
            dimension_semantics=("arbitrary", "arbitrary"),
        ),
    )(x, W1b, b1r, W2b, b2r, Wg, bg2)


# speedup vs baseline: 1.2802x; 1.0726x over previous
import jax
import jax.numpy as jnp
from jax.experimental import pallas as pl
from jax.experimental.pallas import tpu as pltpu
from jax.experimental.pallas import tpu_sc as plsc

D_MODEL = 1024
D_FF = 2048
NUM_EXPERTS = 8
N_TOKENS = 2048
N_ASSIGN = 2 * N_TOKENS
BTS = 256
NB = N_ASSIGN // BTS + NUM_EXPERTS
PAD_N = NB * BTS
GW = 128
SC_W = 16
SC_CW = 16

_vector_mesh = plsc.VectorSubcoreMesh(core_axis_name="c", subcore_axis_name="s")


def _cumsum_rows(a):
    n, m = a.shape
    c = a
    k = 1
    while k < n:
        c = c + jnp.concatenate(
            [jnp.zeros((k, m), a.dtype), c[: n - k]], axis=0)
        k *= 2
    return c


def _cumsum_lanes(a):
    n, m = a.shape
    c = a
    k = 1
    while k < m:
        c = c + jnp.concatenate(
            [jnp.zeros((n, k), a.dtype), c[:, : m - k]], axis=1)
        k *= 2
    return c


def _routing_kernel(x_ref, Wg_ref, bg_ref,
                    pos_ref, gv_ref, gblk_ref, nlive_ref):
    x = x_ref[...]
    logits = jnp.dot(x.astype(jnp.bfloat16), Wg_ref[...].astype(jnp.bfloat16),
                     preferred_element_type=jnp.float32) + bg_ref[0]
    ii = jax.lax.broadcasted_iota(jnp.int32, logits.shape, 1)
    m1 = jnp.max(logits, axis=-1, keepdims=True)
    i1 = jnp.min(jnp.where(logits == m1, ii, NUM_EXPERTS),
                 axis=-1, keepdims=True)
    logits2 = jnp.where(ii == i1, -jnp.inf, logits)
    m2 = jnp.max(logits2, axis=-1, keepdims=True)
    i2 = jnp.min(jnp.where(logits2 == m2, ii, NUM_EXPERTS),
                 axis=-1, keepdims=True)
    g1 = jax.nn.sigmoid(m1 - m2)
    g2 = 1.0 - g1

    one1 = (ii == i1).astype(jnp.float32)
    one2 = (ii == i2).astype(jnp.float32)
    c1 = _cumsum_rows(one1)
    c2 = _cumsum_rows(one2)
    cnt1 = c1[N_TOKENS - 1:, :]
    cnt2 = c2[N_TOKENS - 1:, :]
    counts = cnt1 + cnt2
    nb = jnp.floor((counts + (BTS - 1)) * (1.0 / BTS))
    bb = _cumsum_lanes(nb) - nb
    base = bb * BTS

    rank1 = jnp.sum(jnp.where(ii == i1, c1 - one1 + base, 0.0),
                    axis=-1, keepdims=True)
    rank2 = jnp.sum(jnp.where(ii == i2, c2 - one2 + cnt1 + base, 0.0),
                    axis=-1, keepdims=True)
    pos_ref[0:N_TOKENS, :] = rank1.astype(jnp.int32)
    pos_ref[N_TOKENS:N_ASSIGN, :] = rank2.astype(jnp.int32)

    gv_ref[0:N_TOKENS, :] = jnp.broadcast_to(g1, (N_TOKENS, GW))
    gv_ref[N_TOKENS:N_ASSIGN, :] = jnp.broadcast_to(g2, (N_TOKENS, GW))

    nlive = jnp.sum(nb)
    jj = jax.lax.broadcasted_iota(
        jnp.int32, (NB, NUM_EXPERTS), 0).astype(jnp.float32)
    gblk = jnp.sum(jnp.where(jj >= bb, 1.0, 0.0), axis=-1, keepdims=True) - 1.0
    glast = jnp.sum(jnp.where(jnp.float32(NB - 1) >= bb, 1.0, 0.0)) - 1.0
    glast = jnp.minimum(glast, jnp.float32(NUM_EXPERTS - 1))
    gblk_ref[...] = jnp.minimum(gblk, glast).astype(jnp.int32)
    nlive_ref[...] = nlive.astype(jnp.int32).reshape(1, 1)


def _routing(x, Wg, bg2):
    return pl.pallas_call(
        _routing_kernel,
        grid=(1,),
        in_specs=[
            pl.BlockSpec((N_TOKENS, D_MODEL), lambda i: (0, 0)),
            pl.BlockSpec((D_MODEL, NUM_EXPERTS), lambda i: (0, 0)),
            pl.BlockSpec((1, NUM_EXPERTS), lambda i: (0, 0)),
        ],
        out_specs=[
            pl.BlockSpec((N_ASSIGN, 1), lambda i: (0, 0)),
            pl.BlockSpec((N_ASSIGN, GW), lambda i: (0, 0)),
            pl.BlockSpec((NB, 1), lambda i: (0, 0)),
            pl.BlockSpec((1, 1), lambda i: (0, 0)),
        ],
        out_shape=[
            jax.ShapeDtypeStruct((N_ASSIGN, 1), jnp.int32),
            jax.ShapeDtypeStruct((N_ASSIGN, GW), jnp.float32),
            jax.ShapeDtypeStruct((NB, 1), jnp.int32),
            jax.ShapeDtypeStruct((1, 1), jnp.int32),
        ],
    )(x, Wg, bg2)


def _scatter(x, gv, pos_row):
    @pl.kernel(
        out_type=[
            jax.ShapeDtypeStruct((PAD_N, D_MODEL), jnp.float32),
            jax.ShapeDtypeStruct((PAD_N, GW), jnp.float32),
        ],
        mesh=_vector_mesh,
    )
    def k2(x_hbm, gv_hbm, pos_hbm, xs_hbm, gs_hbm):
        def body(x_vmem, g_vmem, i_vmem):
            i = pl.program_id(0)
            idx = i_vmem[0, pl.ds(i * SC_W, SC_W)]
            pltpu.sync_copy(x_vmem, xs_hbm.at[idx])
            pltpu.sync_copy(g_vmem, gs_hbm.at[idx])

        pltpu.emit_pipeline(
            body,
            grid=(N_ASSIGN // SC_W,),
            in_specs=[
                pl.BlockSpec((SC_W, D_MODEL),
                             index_map=lambda i: (i % (N_TOKENS // SC_W), 0)),
                pl.BlockSpec((SC_W, GW), index_map=lambda i: (i, 0)),
                pl.BlockSpec((1, N_ASSIGN), index_map=lambda i: (0, 0)),
            ],
            out_specs=[],
            core_axis_name=("c", "s"),
            dimension_semantics=(pltpu.PARALLEL,),
        )(x_hbm, gv_hbm, pos_hbm)

    return k2(x, gv, pos_row)


def _ffn_kernel(gblk_ref, nlive_ref, xs_ref, gs_ref,
                W1_ref, b1_ref, W2_ref, b2_ref, os_ref):
    j = pl.program_id(0)

    @pl.when(j < nlive_ref[0])
    def _():
        xs = xs_ref[...]
        h = (jnp.dot(xs, W1_ref[0], preferred_element_type=jnp.float32)
             + b1_ref[0, 0])
        hb = jnp.maximum(h, 0.0)
        o = (jnp.dot(hb, W2_ref[0], preferred_element_type=jnp.float32)
             + b2_ref[0, 0])
        os_ref[...] = o * gs_ref[:, 0:1]


def _ffn(gblk, nlive, xs, gs, W1, b1r, W2, b2r):
    grid_spec = pltpu.PrefetchScalarGridSpec(
        num_scalar_prefetch=2,
        grid=(NB,),
        in_specs=[
            pl.BlockSpec((BTS, D_MODEL), lambda j, gb, nl: (j, 0)),
            pl.BlockSpec((BTS, GW), lambda j, gb, nl: (j, 0)),
            pl.BlockSpec((1, D_MODEL, D_FF), lambda j, gb, nl: (gb[j], 0, 0)),
            pl.BlockSpec((1, 1, D_FF), lambda j, gb, nl: (gb[j], 0, 0)),
            pl.BlockSpec((1, D_FF, D_MODEL), lambda j, gb, nl: (gb[j], 0, 0)),
            pl.BlockSpec((1, 1, D_MODEL), lambda j, gb, nl: (gb[j], 0, 0)),
        ],
        out_specs=pl.BlockSpec((BTS, D_MODEL), lambda j, gb, nl: (j, 0)),
    )
    return pl.pallas_call(
        _ffn_kernel,
        grid_spec=grid_spec,
        out_shape=jax.ShapeDtypeStruct((PAD_N, D_MODEL), jnp.float32),
        compiler_params=pltpu.CompilerParams(
            dimension_semantics=("arbitrary",),
        ),
    )(gblk, nlive, xs, gs, W1, b1r, W2, b2r)


def _combine(os, pos_row):
    @pl.kernel(
        out_type=jax.ShapeDtypeStruct((N_TOKENS, D_MODEL), jnp.float32),
        mesh=_vector_mesh,
        scratch_types=[
            pltpu.VMEM((SC_CW, D_MODEL), jnp.float32),
            pltpu.VMEM((SC_CW, D_MODEL), jnp.float32),
        ],
    )
    def k4(os_hbm, pos_hbm, out_hbm, a_scr, b_scr):
        def body(i_vmem, o_vmem):
            i = pl.program_id(0)
            ia = i_vmem[0, pl.ds(i * SC_CW, SC_CW)]
            ib = i_vmem[0, pl.ds(N_TOKENS + i * SC_CW, SC_CW)]
            pltpu.sync_copy(os_hbm.at[ia], a_scr)
            pltpu.sync_copy(os_hbm.at[ib], b_scr)
            o_vmem[...] = a_scr[...] + b_scr[...]

        pltpu.emit_pipeline(
            body,
            grid=(N_TOKENS // SC_CW,),
            in_specs=[
                pl.BlockSpec((1, N_ASSIGN), index_map=lambda i: (0, 0)),
            ],
            out_specs=[
                pl.BlockSpec((SC_CW, D_MODEL), index_map=lambda i: (i, 0)),
            ],
            core_axis_name=("c", "s"),
            dimension_semantics=(pltpu.PARALLEL,),
        )(pos_hbm, out_hbm)

    return k4(os, pos_row)


def kernel(x, W1, b1, W2, b2, Wg, bg):
    bg2 = bg.reshape(1, NUM_EXPERTS)
    b1r = b1.reshape(NUM_EXPERTS, 1, D_FF)
    b2r = b2.reshape(NUM_EXPERTS, 1, D_MODEL)

    pos, gv, gblk, nlive = _routing(x, Wg, bg2)
    pos_row = pos.reshape(1, N_ASSIGN)
    xs, gs = _scatter(x, gv, pos_row)
    os = _ffn(gblk.reshape(NB), nlive.reshape(1), xs, gs, W1, b1r, W2, b2r)
    return _combine(os, pos_row)


# speedup vs baseline: 1.3017x; 1.0168x over previous
import jax
import jax.numpy as jnp
from jax.experimental import pallas as pl
from jax.experimental.pallas import tpu as pltpu
from jax.experimental.pallas import tpu_sc as plsc

D_MODEL = 1024
D_FF = 2048
NUM_EXPERTS = 8
N_TOKENS = 2048
N_ASSIGN = 2 * N_TOKENS
BTS = 256
NB = N_ASSIGN // BTS + NUM_EXPERTS
PAD_N = NB * BTS
GW = 128
SC_W = 16
SC_CW = 16

_vector_mesh = plsc.VectorSubcoreMesh(core_axis_name="c", subcore_axis_name="s")


def _cumsum_rows(a):
    n, m = a.shape
    c = a
    k = 1
    while k < n:
        c = c + jnp.concatenate(
            [jnp.zeros((k, m), a.dtype), c[: n - k]], axis=0)
        k *= 2
    return c


def _cumsum_lanes(a):
    n, m = a.shape
    c = a
    k = 1
    while k < m:
        c = c + jnp.concatenate(
            [jnp.zeros((n, k), a.dtype), c[:, : m - k]], axis=1)
        k *= 2
    return c


def _routing_kernel(x_ref, Wg_ref, bg_ref,
                    pos_ref, gv_ref, gblk_ref, nlive_ref):
    x = x_ref[...]
    logits = jnp.dot(x.astype(jnp.bfloat16), Wg_ref[...].astype(jnp.bfloat16),
                     preferred_element_type=jnp.float32) + bg_ref[0]
    ii = jax.lax.broadcasted_iota(jnp.int32, logits.shape, 1)
    m1 = jnp.max(logits, axis=-1, keepdims=True)
    i1 = jnp.min(jnp.where(logits == m1, ii, NUM_EXPERTS),
                 axis=-1, keepdims=True)
    logits2 = jnp.where(ii == i1, -jnp.inf, logits)
    m2 = jnp.max(logits2, axis=-1, keepdims=True)
    i2 = jnp.min(jnp.where(logits2 == m2, ii, NUM_EXPERTS),
                 axis=-1, keepdims=True)
    g1 = jax.nn.sigmoid(m1 - m2)
    g2 = 1.0 - g1

    one1 = (ii == i1).astype(jnp.float32)
    one2 = (ii == i2).astype(jnp.float32)
    c1 = _cumsum_rows(one1)
    c2 = _cumsum_rows(one2)
    cnt1 = c1[N_TOKENS - 1:, :]
    cnt2 = c2[N_TOKENS - 1:, :]
    counts = cnt1 + cnt2
    nb = jnp.floor((counts + (BTS - 1)) * (1.0 / BTS))
    bb = _cumsum_lanes(nb) - nb
    base = bb * BTS

    rank1 = jnp.sum(jnp.where(ii == i1, c1 - one1 + base, 0.0),
                    axis=-1, keepdims=True)
    rank2 = jnp.sum(jnp.where(ii == i2, c2 - one2 + cnt1 + base, 0.0),
                    axis=-1, keepdims=True)
    pos_ref[0:N_TOKENS, :] = rank1.astype(jnp.int32)
    pos_ref[N_TOKENS:N_ASSIGN, :] = rank2.astype(jnp.int32)

    gv_ref[0:N_TOKENS, :] = jnp.broadcast_to(g1, (N_TOKENS, GW))
    gv_ref[N_TOKENS:N_ASSIGN, :] = jnp.broadcast_to(g2, (N_TOKENS, GW))

    nlive = jnp.sum(nb)
    jj = jax.lax.broadcasted_iota(
        jnp.int32, (NB, NUM_EXPERTS), 0).astype(jnp.float32)
    gblk = jnp.sum(jnp.where(jj >= bb, 1.0, 0.0), axis=-1, keepdims=True) - 1.0
    glast = jnp.sum(jnp.where(jnp.float32(NB - 1) >= bb, 1.0, 0.0)) - 1.0
    glast = jnp.minimum(glast, jnp.float32(NUM_EXPERTS - 1))
    gblk_ref[...] = jnp.minimum(gblk, glast).astype(jnp.int32)
    nlive_ref[...] = nlive.astype(jnp.int32).reshape(1, 1)


def _routing(x, Wg, bg2):
    return pl.pallas_call(
        _routing_kernel,
        grid=(1,),
        in_specs=[
            pl.BlockSpec((N_TOKENS, D_MODEL), lambda i: (0, 0)),
            pl.BlockSpec((D_MODEL, NUM_EXPERTS), lambda i: (0, 0)),
            pl.BlockSpec((1, NUM_EXPERTS), lambda i: (0, 0)),
        ],
        out_specs=[
            pl.BlockSpec((N_ASSIGN, 1), lambda i: (0, 0)),
            pl.BlockSpec((N_ASSIGN, GW), lambda i: (0, 0)),
            pl.BlockSpec((NB, 1), lambda i: (0, 0)),
            pl.BlockSpec((1, 1), lambda i: (0, 0)),
        ],
        out_shape=[
            jax.ShapeDtypeStruct((N_ASSIGN, 1), jnp.int32),
            jax.ShapeDtypeStruct((N_ASSIGN, GW), jnp.float32),
            jax.ShapeDtypeStruct((NB, 1), jnp.int32),
            jax.ShapeDtypeStruct((1, 1), jnp.int32),
        ],
    )(x, Wg, bg2)


def _scatter(x, gv, pos_row):
    @pl.kernel(
        out_type=[
            jax.ShapeDtypeStruct((PAD_N, D_MODEL), jnp.float32),
            jax.ShapeDtypeStruct((PAD_N, GW), jnp.float32),
        ],
        mesh=_vector_mesh,
        scratch_types=[
            pltpu.SemaphoreType.DMA,
            pltpu.SemaphoreType.DMA,
        ],
    )
    def k2(x_hbm, gv_hbm, pos_hbm, xs_hbm, gs_hbm, sem_x, sem_g):
        def body(x_vmem, g_vmem, i_vmem):
            i = pl.program_id(0)
            idx = i_vmem[0, pl.ds(i * SC_W, SC_W)]
            hx = pltpu.async_copy(x_vmem, xs_hbm.at[idx], sem_x)
            hg = pltpu.async_copy(g_vmem, gs_hbm.at[idx], sem_g)
            hx.wait()
            hg.wait()

        pltpu.emit_pipeline(
            body,
            grid=(N_ASSIGN // SC_W,),
            in_specs=[
                pl.BlockSpec((SC_W, D_MODEL),
                             index_map=lambda i: (i % (N_TOKENS // SC_W), 0)),
                pl.BlockSpec((SC_W, GW), index_map=lambda i: (i, 0)),
                pl.BlockSpec((1, N_ASSIGN), index_map=lambda i: (0, 0)),
            ],
            out_specs=[],
            core_axis_name=("c", "s"),
            dimension_semantics=(pltpu.PARALLEL,),
        )(x_hbm, gv_hbm, pos_hbm)

    return k2(x, gv, pos_row)


def _ffn_kernel(gblk_ref, nlive_ref, xs_ref, gs_ref,
                W1_ref, b1_ref, W2_ref, b2_ref, os_ref):
    j = pl.program_id(0)

    @pl.when(j < nlive_ref[0])
    def _():
        xs = xs_ref[...]
        h = (jnp.dot(xs, W1_ref[0], preferred_element_type=jnp.float32)
             + b1_ref[0, 0])
        hb = jnp.maximum(h, 0.0)
        o = (jnp.dot(hb, W2_ref[0], preferred_element_type=jnp.float32)
             + b2_ref[0, 0])
        os_ref[...] = o * gs_ref[:, 0:1]


def _ffn(gblk, nlive, xs, gs, W1, b1r, W2, b2r):
    grid_spec = pltpu.PrefetchScalarGridSpec(
        num_scalar_prefetch=2,
        grid=(NB,),
        in_specs=[
            pl.BlockSpec((BTS, D_MODEL), lambda j, gb, nl: (j, 0)),
            pl.BlockSpec((BTS, GW), lambda j, gb, nl: (j, 0)),
            pl.BlockSpec((1, D_MODEL, D_FF), lambda j, gb, nl: (gb[j], 0, 0)),
            pl.BlockSpec((1, 1, D_FF), lambda j, gb, nl: (gb[j], 0, 0)),
            pl.BlockSpec((1, D_FF, D_MODEL), lambda j, gb, nl: (gb[j], 0, 0)),
            pl.BlockSpec((1, 1, D_MODEL), lambda j, gb, nl: (gb[j], 0, 0)),
        ],
        out_specs=pl.BlockSpec((BTS, D_MODEL), lambda j, gb, nl: (j, 0)),
    )
    return pl.pallas_call(
        _ffn_kernel,
        grid_spec=grid_spec,
        out_shape=jax.ShapeDtypeStruct((PAD_N, D_MODEL), jnp.float32),
        compiler_params=pltpu.CompilerParams(
            dimension_semantics=("arbitrary",),
        ),
    )(gblk, nlive, xs, gs, W1, b1r, W2, b2r)


def _combine(os, pos_row):
    @pl.kernel(
        out_type=jax.ShapeDtypeStruct((N_TOKENS, D_MODEL), jnp.float32),
        mesh=_vector_mesh,
        scratch_types=[
            pltpu.VMEM((SC_CW, D_MODEL), jnp.float32),
            pltpu.VMEM((SC_CW, D_MODEL), jnp.float32),
            pltpu.SemaphoreType.DMA,
            pltpu.SemaphoreType.DMA,
        ],
    )
    def k4(os_hbm, pos_hbm, out_hbm, a_scr, b_scr, sem_a, sem_b):
        def body(i_vmem, o_vmem):
            i = pl.program_id(0)
            ia = i_vmem[0, pl.ds(i * SC_CW, SC_CW)]
            ib = i_vmem[0, pl.ds(N_TOKENS + i * SC_CW, SC_CW)]
            ha = pltpu.async_copy(os_hbm.at[ia], a_scr, sem_a)
            hb = pltpu.async_copy(os_hbm.at[ib], b_scr, sem_b)
            ha.wait()
            hb.wait()
            o_vmem[...] = a_scr[...] + b_scr[...]

        pltpu.emit_pipeline(
            body,
            grid=(N_TOKENS // SC_CW,),
            in_specs=[
                pl.BlockSpec((1, N_ASSIGN), index_map=lambda i: (0, 0)),
            ],
            out_specs=[
                pl.BlockSpec((SC_CW, D_MODEL), index_map=lambda i: (i, 0)),
            ],
            core_axis_name=("c", "s"),
            dimension_semantics=(pltpu.PARALLEL,),
        )(pos_hbm, out_hbm)

    return k4(os, pos_row)


def kernel(x, W1, b1, W2, b2, Wg, bg):
    bg2 = bg.reshape(1, NUM_EXPERTS)
    b1r = b1.reshape(NUM_EXPERTS, 1, D_FF)
    b2r = b2.reshape(NUM_EXPERTS, 1, D_MODEL)

    pos, gv, gblk, nlive = _routing(x, Wg, bg2)
    pos_row = pos.reshape(1, N_ASSIGN)
    xs, gs = _scatter(x, gv, pos_row)
    os = _ffn(gblk.reshape(NB), nlive.reshape(1), xs, gs, W1, b1r, W2, b2r)
    return _combine(os, pos_row)


# speedup vs baseline: 1.6390x; 1.2592x over previous
import jax
import jax.numpy as jnp
from jax.experimental import pallas as pl
from jax.experimental.pallas import tpu as pltpu
from jax.experimental.pallas import tpu_sc as plsc

D_MODEL = 1024
D_FF = 2048
NUM_EXPERTS = 8
N_TOKENS = 2048
N_ASSIGN = 2 * N_TOKENS
BTS = 256
NB = N_ASSIGN // BTS + NUM_EXPERTS
PAD_N = NB * BTS
GW = 128
SC_W = 16
SC_CW = 16

_vector_mesh = plsc.VectorSubcoreMesh(core_axis_name="c", subcore_axis_name="s")


def _cumsum_rows(a):
    n, m = a.shape
    c = a
    k = 1
    while k < n:
        c = c + jnp.concatenate(
            [jnp.zeros((k, m), a.dtype), c[: n - k]], axis=0)
        k *= 2
    return c


def _cumsum_lanes(a):
    n, m = a.shape
    c = a
    k = 1
    while k < m:
        c = c + jnp.concatenate(
            [jnp.zeros((n, k), a.dtype), c[:, : m - k]], axis=1)
        k *= 2
    return c


def _routing_kernel(x_ref, Wg_ref, bg_ref,
                    pos_ref, gv_ref, gblk_ref, nlive_ref):
    x = x_ref[...]
    logits = jnp.dot(x.astype(jnp.bfloat16), Wg_ref[...].astype(jnp.bfloat16),
                     preferred_element_type=jnp.float32) + bg_ref[0]
    ii = jax.lax.broadcasted_iota(jnp.int32, logits.shape, 1)
    m1 = jnp.max(logits, axis=-1, keepdims=True)
    i1 = jnp.min(jnp.where(logits == m1, ii, NUM_EXPERTS),
                 axis=-1, keepdims=True)
    logits2 = jnp.where(ii == i1, -jnp.inf, logits)
    m2 = jnp.max(logits2, axis=-1, keepdims=True)
    i2 = jnp.min(jnp.where(logits2 == m2, ii, NUM_EXPERTS),
                 axis=-1, keepdims=True)
    g1 = jax.nn.sigmoid(m1 - m2)
    g2 = 1.0 - g1

    one1 = (ii == i1).astype(jnp.float32)
    one2 = (ii == i2).astype(jnp.float32)
    c1 = _cumsum_rows(one1)
    c2 = _cumsum_rows(one2)
    cnt1 = c1[N_TOKENS - 1:, :]
    cnt2 = c2[N_TOKENS - 1:, :]
    counts = cnt1 + cnt2
    nb = jnp.floor((counts + (BTS - 1)) * (1.0 / BTS))
    bb = _cumsum_lanes(nb) - nb
    base = bb * BTS

    rank1 = jnp.sum(jnp.where(ii == i1, c1 - one1 + base, 0.0),
                    axis=-1, keepdims=True)
    rank2 = jnp.sum(jnp.where(ii == i2, c2 - one2 + cnt1 + base, 0.0),
                    axis=-1, keepdims=True)
    pos_ref[0:N_TOKENS, :] = rank1.astype(jnp.int32)
    pos_ref[N_TOKENS:N_ASSIGN, :] = rank2.astype(jnp.int32)

    gv_ref[0:N_TOKENS, :] = jnp.broadcast_to(g1, (N_TOKENS, GW))
    gv_ref[N_TOKENS:N_ASSIGN, :] = jnp.broadcast_to(g2, (N_TOKENS, GW))

    nlive = jnp.sum(nb)
    jj = jax.lax.broadcasted_iota(
        jnp.int32, (NB, NUM_EXPERTS), 0).astype(jnp.float32)
    gblk = jnp.sum(jnp.where(jj >= bb, 1.0, 0.0), axis=-1, keepdims=True) - 1.0
    glast = jnp.sum(jnp.where(jnp.float32(NB - 1) >= bb, 1.0, 0.0)) - 1.0
    glast = jnp.minimum(glast, jnp.float32(NUM_EXPERTS - 1))
    gblk_ref[...] = jnp.minimum(gblk, glast).astype(jnp.int32)
    nlive_ref[...] = nlive.astype(jnp.int32).reshape(1, 1)


def _routing(x, Wg, bg2):
    return pl.pallas_call(
        _routing_kernel,
        grid=(1,),
        in_specs=[
            pl.BlockSpec((N_TOKENS, D_MODEL), lambda i: (0, 0)),
            pl.BlockSpec((D_MODEL, NUM_EXPERTS), lambda i: (0, 0)),
            pl.BlockSpec((1, NUM_EXPERTS), lambda i: (0, 0)),
        ],
        out_specs=[
            pl.BlockSpec((N_ASSIGN, 1), lambda i: (0, 0)),
            pl.BlockSpec((N_ASSIGN, GW), lambda i: (0, 0)),
            pl.BlockSpec((NB, 1), lambda i: (0, 0)),
            pl.BlockSpec((1, 1), lambda i: (0, 0)),
        ],
        out_shape=[
            jax.ShapeDtypeStruct((N_ASSIGN, 1), jnp.int32),
            jax.ShapeDtypeStruct((N_ASSIGN, GW), jnp.float32),
            jax.ShapeDtypeStruct((NB, 1), jnp.int32),
            jax.ShapeDtypeStruct((1, 1), jnp.int32),
        ],
    )(x, Wg, bg2)


def _scatter(x, gv, pos_row):
    @pl.kernel(
        out_type=[
            jax.ShapeDtypeStruct((PAD_N, D_MODEL), jnp.float32),
            jax.ShapeDtypeStruct((PAD_N, GW), jnp.float32),
        ],
        mesh=_vector_mesh,
        scratch_types=[
            pltpu.SemaphoreType.DMA,
            pltpu.SemaphoreType.DMA,
        ],
    )
    def k2(x_hbm, gv_hbm, pos_hbm, xs_hbm, gs_hbm, sem_x, sem_g):
        def body(x_vmem, g_vmem, i_vmem):
            i = pl.program_id(0)
            idx = i_vmem[0, pl.ds(i * SC_W, SC_W)]
            hx = pltpu.async_copy(x_vmem, xs_hbm.at[idx], sem_x)
            hg = pltpu.async_copy(g_vmem, gs_hbm.at[idx], sem_g)
            hx.wait()
            hg.wait()

        pltpu.emit_pipeline(
            body,
            grid=(N_ASSIGN // SC_W,),
            in_specs=[
                pl.BlockSpec((SC_W, D_MODEL),
                             index_map=lambda i: (i % (N_TOKENS // SC_W), 0)),
                pl.BlockSpec((SC_W, GW), index_map=lambda i: (i, 0)),
                pl.BlockSpec((1, N_ASSIGN), index_map=lambda i: (0, 0)),
            ],
            out_specs=[],
            core_axis_name=("c", "s"),
            dimension_semantics=(pltpu.PARALLEL,),
        )(x_hbm, gv_hbm, pos_hbm)

    return k2(x, gv, pos_row)


def _ffn_kernel(gblk_ref, nlive_ref, xs_ref, gs_ref,
                W1_ref, b1_ref, W2_ref, b2_ref, os_ref):
    j = pl.program_id(0)

    @pl.when(j < nlive_ref[0])
    def _():
        xs = xs_ref[...]
        h = (jnp.dot(xs, W1_ref[0], preferred_element_type=jnp.float32)
             + b1_ref[0, 0])
        hb = jnp.maximum(h, 0.0)
        o = (jnp.dot(hb, W2_ref[0], preferred_element_type=jnp.float32)
             + b2_ref[0, 0])
        os_ref[...] = o * gs_ref[:, 0:1]


def _ffn(gblk, nlive, xs, gs, W1, b1r, W2, b2r):
    grid_spec = pltpu.PrefetchScalarGridSpec(
        num_scalar_prefetch=2,
        grid=(NB,),
        in_specs=[
            pl.BlockSpec((BTS, D_MODEL), lambda j, gb, nl: (j, 0)),
            pl.BlockSpec((BTS, GW), lambda j, gb, nl: (j, 0)),
            pl.BlockSpec((1, D_MODEL, D_FF), lambda j, gb, nl: (gb[j], 0, 0)),
            pl.BlockSpec((1, 1, D_FF), lambda j, gb, nl: (gb[j], 0, 0)),
            pl.BlockSpec((1, D_FF, D_MODEL), lambda j, gb, nl: (gb[j], 0, 0)),
            pl.BlockSpec((1, 1, D_MODEL), lambda j, gb, nl: (gb[j], 0, 0)),
        ],
        out_specs=pl.BlockSpec((BTS, D_MODEL), lambda j, gb, nl: (j, 0)),
    )
    return pl.pallas_call(
        _ffn_kernel,
        grid_spec=grid_spec,
        out_shape=jax.ShapeDtypeStruct((PAD_N, D_MODEL), jnp.float32),
        compiler_params=pltpu.CompilerParams(
            dimension_semantics=("arbitrary",),
        ),
    )(gblk, nlive, xs, gs, W1, b1r, W2, b2r)


NUNITS = 32
TOK_PER_U = N_TOKENS // NUNITS
NCHUNK = TOK_PER_U // SC_CW


def _combine(os, pos_row):
    @pl.kernel(
        out_type=jax.ShapeDtypeStruct((N_TOKENS, D_MODEL), jnp.float32),
        mesh=_vector_mesh,
        scratch_types=[
            pltpu.VMEM((1, 2 * TOK_PER_U), jnp.int32),
            pltpu.VMEM((2, SC_CW, D_MODEL), jnp.float32),
            pltpu.VMEM((2, SC_CW, D_MODEL), jnp.float32),
            pltpu.VMEM((SC_CW, D_MODEL), jnp.float32),
            pltpu.SemaphoreType.DMA((2,)),
            pltpu.SemaphoreType.DMA((2,)),
            pltpu.SemaphoreType.DMA,
            pltpu.SemaphoreType.DMA,
            pltpu.SemaphoreType.DMA,
        ],
    )
    def k4(os_hbm, pos_hbm, out_hbm, idx_scr, a_scr, b_scr, o_scr,
           sem_a, sem_b, sem_o, sem_i1, sem_i2):
        c = jax.lax.axis_index("c")
        s = jax.lax.axis_index("s")
        u = c * (NUNITS // 2) + s
        t0 = u * TOK_PER_U
        h1 = pltpu.async_copy(pos_hbm.at[:, pl.ds(t0, TOK_PER_U)],
                              idx_scr.at[:, 0:TOK_PER_U], sem_i1)
        h2 = pltpu.async_copy(pos_hbm.at[:, pl.ds(N_TOKENS + t0, TOK_PER_U)],
                              idx_scr.at[:, TOK_PER_U:2 * TOK_PER_U], sem_i2)
        h1.wait()
        h2.wait()

        def start_gather(k, slot):
            ia = idx_scr[0, pl.ds(k * SC_CW, SC_CW)]
            ib = idx_scr[0, pl.ds(TOK_PER_U + k * SC_CW, SC_CW)]
            ha = pltpu.async_copy(os_hbm.at[ia], a_scr.at[slot],
                                  sem_a.at[slot])
            hb = pltpu.async_copy(os_hbm.at[ib], b_scr.at[slot],
                                  sem_b.at[slot])
            return ha, hb

        gh = [None, None]
        oh = None
        gh[0] = start_gather(0, 0)
        for k in range(NCHUNK):
            slot = k % 2
            nslot = (k + 1) % 2
            if k + 1 < NCHUNK:
                gh[nslot] = start_gather(k + 1, nslot)
            ha, hb = gh[slot]
            ha.wait()
            hb.wait()
            if oh is not None:
                oh.wait()

            @pl.loop(0, SC_CW)
            def _(r):
                for cc in range(0, D_MODEL, 128):
                    o_scr[r, pl.ds(cc, 128)] = (
                        a_scr[slot, r, pl.ds(cc, 128)]
                        + b_scr[slot, r, pl.ds(cc, 128)])
            oh = pltpu.async_copy(
                o_scr, out_hbm.at[pl.ds(t0 + k * SC_CW, SC_CW), :], sem_o)
        oh.wait()

    return k4(os, pos_row)


def kernel(x, W1, b1, W2, b2, Wg, bg):
    bg2 = bg.reshape(1, NUM_EXPERTS)
    b1r = b1.reshape(NUM_EXPERTS, 1, D_FF)
    b2r = b2.reshape(NUM_EXPERTS, 1, D_MODEL)

    pos, gv, gblk, nlive = _routing(x, Wg, bg2)
    pos_row = pos.reshape(1, N_ASSIGN)
    xs, gs = _scatter(x, gv, pos_row)
    os = _ffn(gblk.reshape(NB), nlive.reshape(1), xs, gs, W1, b1r, W2, b2r)
    return _combine(os, pos_row)
